# baseline (speedup 1.0000x reference)
import functools
import jax
import jax.numpy as jnp
from jax import lax
from jax.experimental import pallas as pl
from jax.experimental.pallas import tpu as pltpu
from jax.experimental.pallas import tpu_sc as plsc

NB = 32
DIN = 3200
DH = 8192
DO = 4096
PW = 128
NC = 2
NS = 16
L = 16
BPC = NB // NC
G = 8
NPASS = BPC // G
DOUT = 256
EPS = 1e-12


def _resilu(v):
    sig = 1.0 / (1.0 + jnp.exp(-v))
    return jnp.clip(v * sig, 0.0, 1.0)


def _rsqrt(x):
    i = lax.bitcast_convert_type(x, jnp.int32)
    i = jnp.int32(0x5F3759DF) - (i >> 1)
    y = lax.bitcast_convert_type(i, jnp.float32)
    for _ in range(4):
        y = y * (1.5 - 0.5 * x * y * y)
    return y


def _group_major(a):
    og = a.shape[0] // L
    return a.reshape(og, L, PW).transpose(0, 2, 1).reshape(og, PW * L)


def _pack_group_major(sel):
    og = sel.shape[0] // L
    t = sel.reshape(og, L, PW).transpose(0, 2, 1)
    packed = t[:, 0::2, :] | (t[:, 1::2, :] << 16)
    return packed.reshape(og, PW * L // 2)


def _body(x, sel1, w1, b1, g1, be1, sel2, w2, b2, g2, be2, sel3, w3, b3,
          out, h1, h2,
          table, idxb, wb, biasb, gsc, besc, outb, semA, semB, semT, semG):
    c = lax.axis_index("c")
    s = lax.axis_index("s")
    cbase = c * BPC
    sems = (semA, semB)
    BLK = PW * L
    BLKI = PW * L

    def gather_layer(src_hbm, src_w, selT, wT, bias_hbm, U, dst_hbm, final,
                     GL=G):
        ngrp = U // L
        npass = BPC // GL
        pltpu.sync_copy(bias_hbm.at[pl.ds(s * U, U)], biasb.at[pl.ds(0, U)])

        def start_grp(g, slot):
            gg = s * ngrp + jnp.minimum(g, ngrp - 1)
            pltpu.async_copy(selT.at[gg], idxb.at[pl.ds(slot * BLKI, BLKI)],
                             sems[slot])
            pltpu.async_copy(wT.at[gg], wb.at[pl.ds(slot * BLK, BLK)],
                             sems[slot])

        def wait_grp(slot):
            pltpu.make_async_copy(
                selT.at[0], idxb.at[pl.ds(slot * BLKI, BLKI)],
                sems[slot]).wait()
            pltpu.make_async_copy(
                wT.at[0], wb.at[pl.ds(slot * BLK, BLK)], sems[slot]).wait()

        def compute_grp(g, slot):
            offi = slot * BLKI
            offw = slot * BLK
            zero = jnp.zeros((L,), jnp.float32)

            def p_body(p, accs):
                idx = idxb[pl.ds(offi + p * L, L)]
                rw = _resilu(wb[pl.ds(offw + p * L, L)])
                return tuple(
                    accs[b]
                    + plsc.load_gather(table, [idx + jnp.int32(b * src_w)])
                    * rw
                    for b in range(GL))

            accs = plsc.parallel_loop(
                0, PW, unroll=4, carry=(zero,) * GL)(p_body)
            bv = biasb[pl.ds(g * L, L)]
            for b in range(GL):
                outb[pl.ds(b * U + g * L, L)] = _resilu(accs[b] - bv)

        for pss in range(npass):
            bbase = cbase + pss * GL
            start_grp(0, 0)
            hs = []
            for b in range(GL):
                hs.append(pltpu.async_copy(
                    src_hbm.at[bbase + b, pl.ds(0, src_w)],
                    table.at[pl.ds(b * src_w, src_w)], semT))
            for h in hs:
                h.wait()

            def two_groups(gp, _):
                g0 = 2 * gp
                start_grp(g0 + 1, 1)
                wait_grp(0)
                compute_grp(g0, 0)
                start_grp(g0 + 2, 0)
                wait_grp(1)
                compute_grp(g0 + 1, 1)
                return 0

            lax.fori_loop(0, ngrp // 2, two_groups, 0)
            wait_grp(0)

            if not final:
                ohs = []
                for b in range(GL):
                    ohs.append(pltpu.async_copy(
                        outb.at[pl.ds(b * U, U)],
                        dst_hbm.at[bbase + b, pl.ds(s * U, U)], semT))
                for h in ohs:
                    h.wait()
            else:
                lanes = lax.iota(jnp.int32, L)
                for b in range(GL):
                    acc = jnp.full((L,), -8.0, jnp.float32)
                    for l in range(L):
                        gi = b * U + lanes * L + l
                        acc = acc + plsc.load_gather(outb, [gi])
                    outb[pl.ds(b * U, L)] = acc
                    pltpu.sync_copy(outb.at[pl.ds(b * U, L)],
                                    out.at[bbase + b, pl.ds(s * L, L)])

    def norm_layer(h_hbm, g_hbm, be_hbm):
        b = cbase + s
        hg = pltpu.async_copy(g_hbm, gsc, semG)
        hb = pltpu.async_copy(be_hbm, besc, semG)
        pltpu.sync_copy(h_hbm.at[b], table.at[pl.ds(0, DH)])

        def acc_body(i, carry):
            sm, sq = carry
            v = table[pl.ds(i * L, L)]
            return (sm + v, sq + v * v)

        zero = jnp.zeros((L,), jnp.float32)
        sm, sq = lax.fori_loop(0, DH // L, acc_body, (zero, zero))
        tot = jnp.sum(sm)
        tot2 = jnp.sum(sq)
        mean = tot * (1.0 / DH)
        var = tot2 * (1.0 / DH) - mean * mean
        istd = _rsqrt(var + EPS)
        hg.wait()
        hb.wait()

        def n_body(i, _):
            v = table[pl.ds(i * L, L)]
            gv = gsc[pl.ds(i * L, L)]
            bev = besc[pl.ds(i * L, L)]
            table[pl.ds(i * L, L)] = (v - mean) * istd * gv + bev
            return 0

        lax.fori_loop(0, DH // L, n_body, 0)
        pltpu.sync_copy(table.at[pl.ds(0, DH)], h_hbm.at[b])

    gather_layer(x, DIN, sel1, w1, b1, DH // NS, h1, False, GL=16)
    plsc.subcore_barrier()
    norm_layer(h1, g1, be1)
    plsc.subcore_barrier()
    gather_layer(h1, DH, sel2, w2, b2, DH // NS, h2, False)
    plsc.subcore_barrier()
    norm_layer(h2, g2, be2)
    plsc.subcore_barrier()
    gather_layer(h2, DH, sel3, w3, b3, DO // NS, None, True)


@jax.jit
def _run(x, sel1T, w1T, b1, g1, be1, sel2T, w2T, b2, g2, be2, sel3T, w3T, b3):
    mesh = plsc.VectorSubcoreMesh(core_axis_name="c", subcore_axis_name="s",
                                  num_cores=NC, num_subcores=NS)
    f = pl.kernel(
        _body,
        out_type=[
            jax.ShapeDtypeStruct((NB, DOUT), jnp.float32),
            jax.ShapeDtypeStruct((NB, DH), jnp.float32),
            jax.ShapeDtypeStruct((NB, DH), jnp.float32),
        ],
        mesh=mesh,
        compiler_params=pltpu.CompilerParams(needs_layout_passes=False),
        scratch_types=[
            pltpu.VMEM((G * DH,), jnp.float32),
            pltpu.VMEM((2 * PW * L,), jnp.int32),
            pltpu.VMEM((2 * PW * L,), jnp.float32),
            pltpu.VMEM((DH // NS,), jnp.float32),
            pltpu.VMEM((DH,), jnp.float32),
            pltpu.VMEM((DH,), jnp.float32),
            pltpu.VMEM((BPC * (DH // NS),), jnp.float32),
            pltpu.SemaphoreType.DMA,
            pltpu.SemaphoreType.DMA,
            pltpu.SemaphoreType.DMA,
            pltpu.SemaphoreType.DMA,
        ],
    )
    o, _, _ = f(x, sel1T, w1T, b1, g1, be1, sel2T, w2T, b2, g2, be2,
                sel3T, w3T, b3)
    return o


def kernel(x, sel1, w1, b1, g1, be1, sel2, w2, b2, g2, be2, sel3, w3, b3):
    sel1T = _group_major(sel1)
    w1T = _group_major(w1)
    sel2T = _group_major(sel2)
    w2T = _group_major(w2)
    sel3T = _group_major(sel3)
    w3T = _group_major(w3)
    return _run(x, sel1T, w1T, b1, g1, be1, sel2T, w2T, b2, g2, be2,
                sel3T, w3T, b3)


# speedup vs baseline: 1.0334x; 1.0334x over previous
import functools
import jax
import jax.numpy as jnp
from jax import lax
from jax.experimental import pallas as pl
from jax.experimental.pallas import tpu as pltpu
from jax.experimental.pallas import tpu_sc as plsc

NB = 32
DIN = 3200
DH = 8192
DO = 4096
PW = 128
NC = 2
NS = 16
L = 16
BPC = NB // NC
G = 8
NPASS = BPC // G
DOUT = 256
EPS = 1e-12


def _resilu(v):
    sig = 1.0 / (1.0 + jnp.exp(-v))
    return jnp.clip(v * sig, 0.0, 1.0)


def _rsqrt(x):
    i = lax.bitcast_convert_type(x, jnp.int32)
    i = jnp.int32(0x5F3759DF) - (i >> 1)
    y = lax.bitcast_convert_type(i, jnp.float32)
    for _ in range(4):
        y = y * (1.5 - 0.5 * x * y * y)
    return y


def _group_major(a):
    og = a.shape[0] // L
    return a.reshape(og, L, PW).transpose(0, 2, 1).reshape(og, PW * L)


def _pack_group_major(sel):
    og = sel.shape[0] // L
    t = sel.reshape(og, L, PW).transpose(0, 2, 1)
    packed = t[:, 0::2, :] | (t[:, 1::2, :] << 16)
    return packed.reshape(og, PW * L // 2)


def _body(x, sel1, w1, b1, g1, be1, sel2, w2, b2, g2, be2, sel3, w3, b3,
          out,
          h1, h2, table, idxb, wb, biasb, gsc, besc, outb,
          semA, semB, semT, semG):
    c = lax.axis_index("c")
    s = lax.axis_index("s")
    cbase = c * BPC
    sems = (semA, semB)
    BLK = PW * L
    BLKI = PW * L

    def gather_layer(src_hbm, src_w, selT, wT, bias_hbm, U, dst_hbm, final,
                     GL=G):
        ngrp = U // L
        npass = BPC // GL
        pltpu.sync_copy(bias_hbm.at[pl.ds(s * U, U)], biasb.at[pl.ds(0, U)])

        def start_grp(g, slot):
            gg = s * ngrp + jnp.minimum(g, ngrp - 1)
            pltpu.async_copy(selT.at[gg], idxb.at[pl.ds(slot * BLKI, BLKI)],
                             sems[slot])
            pltpu.async_copy(wT.at[gg], wb.at[pl.ds(slot * BLK, BLK)],
                             sems[slot])

        def wait_grp(slot):
            pltpu.make_async_copy(
                selT.at[0], idxb.at[pl.ds(slot * BLKI, BLKI)],
                sems[slot]).wait()
            pltpu.make_async_copy(
                wT.at[0], wb.at[pl.ds(slot * BLK, BLK)], sems[slot]).wait()

        def compute_grp(g, slot):
            offi = slot * BLKI
            offw = slot * BLK
            zero = jnp.zeros((L,), jnp.float32)

            def p_body(p, accs):
                idx = idxb[pl.ds(offi + p * L, L)]
                rw = _resilu(wb[pl.ds(offw + p * L, L)])
                return tuple(
                    accs[b]
                    + plsc.load_gather(table, [idx + jnp.int32(b * src_w)])
                    * rw
                    for b in range(GL))

            accs = plsc.parallel_loop(
                0, PW, unroll=2, carry=(zero,) * GL)(p_body)
            bv = biasb[pl.ds(g * L, L)]
            for b in range(GL):
                outb[pl.ds(b * U + g * L, L)] = _resilu(accs[b] - bv)

        for pss in range(npass):
            bbase = cbase + pss * GL
            start_grp(0, 0)
            hs = []
            for b in range(GL):
                hs.append(pltpu.async_copy(
                    src_hbm.at[bbase + b, pl.ds(0, src_w)],
                    table.at[pl.ds(b * src_w, src_w)], semT))
            for h in hs:
                h.wait()

            def two_groups(gp, _):
                g0 = 2 * gp
                start_grp(g0 + 1, 1)
                wait_grp(0)
                compute_grp(g0, 0)
                start_grp(g0 + 2, 0)
                wait_grp(1)
                compute_grp(g0 + 1, 1)
                return 0

            lax.fori_loop(0, ngrp // 2, two_groups, 0)
            wait_grp(0)

            if not final:
                ohs = []
                for b in range(GL):
                    ohs.append(pltpu.async_copy(
                        outb.at[pl.ds(b * U, U)],
                        dst_hbm.at[bbase + b, pl.ds(s * U, U)], semT))
                for h in ohs:
                    h.wait()
            else:
                lanes = lax.iota(jnp.int32, L)
                for b in range(GL):
                    acc = jnp.full((L,), -8.0, jnp.float32)
                    for l in range(L):
                        gi = b * U + lanes * L + l
                        acc = acc + plsc.load_gather(outb, [gi])
                    outb[pl.ds(b * U, L)] = acc
                    pltpu.sync_copy(outb.at[pl.ds(b * U, L)],
                                    out.at[bbase + b, pl.ds(s * L, L)])

    def norm_layer(h_hbm, g_hbm, be_hbm):
        b = cbase + s
        hg = pltpu.async_copy(g_hbm, gsc, semG)
        hb = pltpu.async_copy(be_hbm, besc, semG)
        pltpu.sync_copy(h_hbm.at[b], table.at[pl.ds(0, DH)])

        def acc_body(i, carry):
            sm, sq = carry
            v = table[pl.ds(i * L, L)]
            return (sm + v, sq + v * v)

        zero = jnp.zeros((L,), jnp.float32)
        sm, sq = lax.fori_loop(0, DH // L, acc_body, (zero, zero))
        tot = jnp.sum(sm)
        tot2 = jnp.sum(sq)
        mean = tot * (1.0 / DH)
        var = tot2 * (1.0 / DH) - mean * mean
        istd = _rsqrt(var + EPS)
        hg.wait()
        hb.wait()

        def n_body(i, _):
            v = table[pl.ds(i * L, L)]
            gv = gsc[pl.ds(i * L, L)]
            bev = besc[pl.ds(i * L, L)]
            table[pl.ds(i * L, L)] = (v - mean) * istd * gv + bev
            return 0

        lax.fori_loop(0, DH // L, n_body, 0)
        pltpu.sync_copy(table.at[pl.ds(0, DH)], h_hbm.at[b])

    gather_layer(x, DIN, sel1, w1, b1, DH // NS, h1, False, GL=16)
    plsc.subcore_barrier()
    norm_layer(h1, g1, be1)
    plsc.subcore_barrier()
    gather_layer(h1, DH, sel2, w2, b2, DH // NS, h2, False)
    plsc.subcore_barrier()
    norm_layer(h2, g2, be2)
    plsc.subcore_barrier()
    gather_layer(h2, DH, sel3, w3, b3, DO // NS, None, True)


@jax.jit
def _run(x, sel1T, w1T, b1, g1, be1, sel2T, w2T, b2, g2, be2, sel3T, w3T, b3):
    mesh = plsc.VectorSubcoreMesh(core_axis_name="c", subcore_axis_name="s",
                                  num_cores=NC, num_subcores=NS)
    f = pl.kernel(
        _body,
        out_type=[
            jax.ShapeDtypeStruct((NB, DOUT), jnp.float32),
        ],
        mesh=mesh,
        compiler_params=pltpu.CompilerParams(needs_layout_passes=False),
        scratch_types=[
            pltpu.HBM((NB, DH), jnp.float32),
            pltpu.HBM((NB, DH), jnp.float32),
            pltpu.VMEM((G * DH,), jnp.float32),
            pltpu.VMEM((2 * PW * L,), jnp.int32),
            pltpu.VMEM((2 * PW * L,), jnp.float32),
            pltpu.VMEM((DH // NS,), jnp.float32),
            pltpu.VMEM((DH,), jnp.float32),
            pltpu.VMEM((DH,), jnp.float32),
            pltpu.VMEM((BPC * (DH // NS),), jnp.float32),
            pltpu.SemaphoreType.DMA,
            pltpu.SemaphoreType.DMA,
            pltpu.SemaphoreType.DMA,
            pltpu.SemaphoreType.DMA,
        ],
    )
    (o,) = f(x, sel1T, w1T, b1, g1, be1, sel2T, w2T, b2, g2, be2,
             sel3T, w3T, b3)
    return o


def kernel(x, sel1, w1, b1, g1, be1, sel2, w2, b2, g2, be2, sel3, w3, b3):
    sel1T = _group_major(sel1)
    w1T = _group_major(w1)
    sel2T = _group_major(sel2)
    w2T = _group_major(w2)
    sel3T = _group_major(sel3)
    w3T = _group_major(w3)
    return _run(x, sel1T, w1T, b1, g1, be1, sel2T, w2T, b2, g2, be2,
                sel3T, w3T, b3)


# speedup vs baseline: 1.0614x; 1.0270x over previous
import functools
import jax
import jax.numpy as jnp
from jax import lax
from jax.experimental import pallas as pl
from jax.experimental.pallas import tpu as pltpu
from jax.experimental.pallas import tpu_sc as plsc

NB = 32
DIN = 3200
DH = 8192
DO = 4096
PW = 128
NC = 2
NS = 16
L = 16
BPC = NB // NC
G = 8
NPASS = BPC // G
DOUT = 256
EPS = 1e-12


def _resilu(v):
    sig = 1.0 / (1.0 + jnp.exp(-v))
    return jnp.clip(v * sig, 0.0, 1.0)


def _rsqrt(x):
    i = lax.bitcast_convert_type(x, jnp.int32)
    i = jnp.int32(0x5F3759DF) - (i >> 1)
    y = lax.bitcast_convert_type(i, jnp.float32)
    for _ in range(4):
        y = y * (1.5 - 0.5 * x * y * y)
    return y


def _group_major(a):
    og = a.shape[0] // L
    return a.reshape(og, L, PW).transpose(0, 2, 1).reshape(og, PW * L)


def _pack_group_major(sel):
    og = sel.shape[0] // L
    t = sel.reshape(og, L, PW).transpose(0, 2, 1)
    packed = t[:, 0::2, :] | (t[:, 1::2, :] << 16)
    return packed.reshape(og, PW * L // 2)


def _body(x, sel1, w1, b1, g1, be1, sel2, w2, b2, g2, be2, sel3, w3, b3,
          out,
          h1, h2, table, idxb, wb, biasb, gsc, besc, outb,
          semA, semB, semT, semG):
    c = lax.axis_index("c")
    s = lax.axis_index("s")
    cbase = c * BPC
    sems = (semA, semB)
    BLK = PW * L
    BLKI = PW * L

    def gather_layer(src_hbm, src_w, selT, wT, bias_hbm, U, dst_hbm, final,
                     GL=G):
        ngrp = U // L
        npass = BPC // GL
        pltpu.sync_copy(bias_hbm.at[pl.ds(s * U, U)], biasb.at[pl.ds(0, U)])

        def start_grp(g, slot):
            gg = s * ngrp + jnp.minimum(g, ngrp - 1)
            pltpu.async_copy(selT.at[gg], idxb.at[pl.ds(slot * BLKI, BLKI)],
                             sems[slot])
            pltpu.async_copy(wT.at[gg], wb.at[pl.ds(slot * BLK, BLK)],
                             sems[slot])

        def wait_grp(slot):
            pltpu.make_async_copy(
                selT.at[0], idxb.at[pl.ds(slot * BLKI, BLKI)],
                sems[slot]).wait()
            pltpu.make_async_copy(
                wT.at[0], wb.at[pl.ds(slot * BLK, BLK)], sems[slot]).wait()

        def compute_grp(g, slot):
            offi = slot * BLKI
            offw = slot * BLK
            zero = jnp.zeros((L,), jnp.float32)

            def p_body(p, accs):
                idx = idxb[pl.ds(offi + p * L, L)]
                rw = _resilu(wb[pl.ds(offw + p * L, L)])
                return tuple(
                    accs[b]
                    + plsc.load_gather(table, [idx + jnp.int32(b * src_w)])
                    * rw
                    for b in range(GL))

            accs = plsc.parallel_loop(
                0, PW, unroll=2, carry=(zero,) * GL)(p_body)
            bv = biasb[pl.ds(g * L, L)]
            for b in range(GL):
                outb[pl.ds(b * U + g * L, L)] = _resilu(accs[b] - bv)

        for pss in range(npass):
            bbase = cbase + pss * GL
            start_grp(0, 0)
            hs = []
            for b in range(GL):
                hs.append(pltpu.async_copy(
                    src_hbm.at[bbase + b, pl.ds(0, src_w)],
                    table.at[pl.ds(b * src_w, src_w)], semT))
            for h in hs:
                h.wait()

            def two_groups(gp, _):
                g0 = 2 * gp
                start_grp(g0 + 1, 1)
                wait_grp(0)
                compute_grp(g0, 0)
                start_grp(g0 + 2, 0)
                wait_grp(1)
                compute_grp(g0 + 1, 1)
                return 0

            lax.fori_loop(0, ngrp // 2, two_groups, 0)
            wait_grp(0)

            if not final:
                ohs = []
                for b in range(GL):
                    ohs.append(pltpu.async_copy(
                        outb.at[pl.ds(b * U, U)],
                        dst_hbm.at[bbase + b, pl.ds(s * U, U)], semT))
                for h in ohs:
                    h.wait()
            else:
                lanes = lax.iota(jnp.int32, L)
                for b in range(GL):
                    acc = jnp.full((L,), -8.0, jnp.float32)
                    for l in range(L):
                        gi = b * U + lanes * L + l
                        acc = acc + plsc.load_gather(outb, [gi])
                    outb[pl.ds(b * U, L)] = acc
                    pltpu.sync_copy(outb.at[pl.ds(b * U, L)],
                                    out.at[bbase + b, pl.ds(s * L, L)])

    def norm_layer(h_hbm, g_hbm, be_hbm):
        b = cbase + s
        hg = pltpu.async_copy(g_hbm, gsc, semG)
        hb = pltpu.async_copy(be_hbm, besc, semG)
        pltpu.sync_copy(h_hbm.at[b], table.at[pl.ds(0, DH)])

        def acc_body(i, carry):
            sm, sq = carry
            v = table[pl.ds(i * L, L)]
            return (sm + v, sq + v * v)

        zero = jnp.zeros((L,), jnp.float32)
        sm, sq = plsc.parallel_loop(
            0, DH // L, unroll=4, carry=(zero, zero))(acc_body)
        tot = jnp.sum(sm)
        tot2 = jnp.sum(sq)
        mean = tot * (1.0 / DH)
        var = tot2 * (1.0 / DH) - mean * mean
        istd = _rsqrt(var + EPS)
        hg.wait()
        hb.wait()

        def n_body(i):
            v = table[pl.ds(i * L, L)]
            gv = gsc[pl.ds(i * L, L)]
            bev = besc[pl.ds(i * L, L)]
            table[pl.ds(i * L, L)] = (v - mean) * istd * gv + bev

        plsc.parallel_loop(0, DH // L, unroll=4)(n_body)
        pltpu.sync_copy(table.at[pl.ds(0, DH)], h_hbm.at[b])

    gather_layer(x, DIN, sel1, w1, b1, DH // NS, h1, False, GL=16)
    plsc.subcore_barrier()
    norm_layer(h1, g1, be1)
    plsc.subcore_barrier()
    gather_layer(h1, DH, sel2, w2, b2, DH // NS, h2, False)
    plsc.subcore_barrier()
    norm_layer(h2, g2, be2)
    plsc.subcore_barrier()
    gather_layer(h2, DH, sel3, w3, b3, DO // NS, None, True)


@jax.jit
def _run(x, sel1T, w1T, b1, g1, be1, sel2T, w2T, b2, g2, be2, sel3T, w3T, b3):
    mesh = plsc.VectorSubcoreMesh(core_axis_name="c", subcore_axis_name="s",
                                  num_cores=NC, num_subcores=NS)
    f = pl.kernel(
        _body,
        out_type=[
            jax.ShapeDtypeStruct((NB, DOUT), jnp.float32),
        ],
        mesh=mesh,
        compiler_params=pltpu.CompilerParams(needs_layout_passes=False),
        scratch_types=[
            pltpu.HBM((NB, DH), jnp.float32),
            pltpu.HBM((NB, DH), jnp.float32),
            pltpu.VMEM((G * DH,), jnp.float32),
            pltpu.VMEM((2 * PW * L,), jnp.int32),
            pltpu.VMEM((2 * PW * L,), jnp.float32),
            pltpu.VMEM((DH // NS,), jnp.float32),
            pltpu.VMEM((DH,), jnp.float32),
            pltpu.VMEM((DH,), jnp.float32),
            pltpu.VMEM((BPC * (DH // NS),), jnp.float32),
            pltpu.SemaphoreType.DMA,
            pltpu.SemaphoreType.DMA,
            pltpu.SemaphoreType.DMA,
            pltpu.SemaphoreType.DMA,
        ],
    )
    (o,) = f(x, sel1T, w1T, b1, g1, be1, sel2T, w2T, b2, g2, be2,
             sel3T, w3T, b3)
    return o


def kernel(x, sel1, w1, b1, g1, be1, sel2, w2, b2, g2, be2, sel3, w3, b3):
    sel1T = _group_major(sel1)
    w1T = _group_major(w1)
    sel2T = _group_major(sel2)
    w2T = _group_major(w2)
    sel3T = _group_major(sel3)
    w3T = _group_major(w3)
    return _run(x, sel1T, w1T, b1, g1, be1, sel2T, w2T, b2, g2, be2,
                sel3T, w3T, b3)


# speedup vs baseline: 1.0623x; 1.0008x over previous
import functools
import jax
import jax.numpy as jnp
from jax import lax
from jax.experimental import pallas as pl
from jax.experimental.pallas import tpu as pltpu
from jax.experimental.pallas import tpu_sc as plsc

NB = 32
DIN = 3200
DH = 8192
DO = 4096
PW = 128
NC = 2
NS = 16
L = 16
BPC = NB // NC
G = 8
NPASS = BPC // G
DOUT = 256
EPS = 1e-12


def _resilu(v):
    sig = 1.0 / (1.0 + jnp.exp(-v))
    return jnp.clip(v * sig, 0.0, 1.0)


def _rsqrt(x):
    i = lax.bitcast_convert_type(x, jnp.int32)
    i = jnp.int32(0x5F3759DF) - (i >> 1)
    y = lax.bitcast_convert_type(i, jnp.float32)
    for _ in range(4):
        y = y * (1.5 - 0.5 * x * y * y)
    return y


def _group_major(a):
    og = a.shape[0] // L
    return a.reshape(og, L, PW).transpose(0, 2, 1).reshape(og, PW * L)


def _pack_group_major(sel):
    og = sel.shape[0] // L
    t = sel.reshape(og, L, PW).transpose(0, 2, 1)
    packed = t[:, 0::2, :] | (t[:, 1::2, :] << 16)
    return packed.reshape(og, PW * L // 2)


def _body(x, sel1, w1, b1, g1, be1, sel2, w2, b2, g2, be2, sel3, w3, b3,
          out,
          h1, h2, table, idxb, wb, biasb, gsc, besc, outb,
          semA, semB, semT, semG):
    c = lax.axis_index("c")
    s = lax.axis_index("s")
    cbase = c * BPC
    sems = (semA, semB)
    BLK = PW * L
    BLKI = PW * L

    def gather_layer(src_hbm, src_w, selT, wT, bias_hbm, U, dst_hbm, final,
                     GL=G):
        ngrp = U // L
        npass = BPC // GL
        pltpu.sync_copy(bias_hbm.at[pl.ds(s * U, U)], biasb.at[pl.ds(0, U)])

        def start_grp(g, slot):
            gg = s * ngrp + jnp.minimum(g, ngrp - 1)
            pltpu.async_copy(selT.at[gg], idxb.at[pl.ds(slot * BLKI, BLKI)],
                             sems[slot])
            pltpu.async_copy(wT.at[gg], wb.at[pl.ds(slot * BLK, BLK)],
                             sems[slot])

        def wait_grp(slot):
            pltpu.make_async_copy(
                selT.at[0], idxb.at[pl.ds(slot * BLKI, BLKI)],
                sems[slot]).wait()
            pltpu.make_async_copy(
                wT.at[0], wb.at[pl.ds(slot * BLK, BLK)], sems[slot]).wait()

        def compute_grp(g, slot):
            offi = slot * BLKI
            offw = slot * BLK
            zero = jnp.zeros((L,), jnp.float32)

            def p_body(p, accs):
                idx = idxb[pl.ds(offi + p * L, L)]
                rw = _resilu(wb[pl.ds(offw + p * L, L)])
                return tuple(
                    accs[b]
                    + plsc.load_gather(table, [idx + jnp.int32(b * src_w)])
                    * rw
                    for b in range(GL))

            accs = plsc.parallel_loop(
                0, PW, unroll=2, carry=(zero,) * GL)(p_body)
            bv = biasb[pl.ds(g * L, L)]
            for b in range(GL):
                outb[pl.ds(b * U + g * L, L)] = _resilu(accs[b] - bv)

        def fire_tables(pss):
            bbase = cbase + pss * GL
            return [pltpu.async_copy(
                src_hbm.at[bbase + b, pl.ds(0, src_w)],
                table.at[pl.ds(b * src_w, src_w)], semT)
                for b in range(GL)]

        ths = fire_tables(0)
        for pss in range(npass):
            bbase = cbase + pss * GL
            start_grp(0, 0)
            for h in ths:
                h.wait()

            def two_groups(gp, _):
                g0 = 2 * gp
                start_grp(g0 + 1, 1)
                wait_grp(0)
                compute_grp(g0, 0)
                start_grp(g0 + 2, 0)
                wait_grp(1)
                compute_grp(g0 + 1, 1)
                return 0

            lax.fori_loop(0, ngrp // 2, two_groups, 0)
            wait_grp(0)
            if pss + 1 < npass:
                ths = fire_tables(pss + 1)

            if not final:
                ohs = []
                for b in range(GL):
                    ohs.append(pltpu.async_copy(
                        outb.at[pl.ds(b * U, U)],
                        dst_hbm.at[bbase + b, pl.ds(s * U, U)], semT))
                for h in ohs:
                    h.wait()
            else:
                lanes = lax.iota(jnp.int32, L)
                for b in range(GL):
                    acc = jnp.full((L,), -8.0, jnp.float32)
                    for l in range(L):
                        gi = b * U + lanes * L + l
                        acc = acc + plsc.load_gather(outb, [gi])
                    outb[pl.ds(b * U, L)] = acc
                    pltpu.sync_copy(outb.at[pl.ds(b * U, L)],
                                    out.at[bbase + b, pl.ds(s * L, L)])

    def norm_layer(h_hbm, g_hbm, be_hbm):
        b = cbase + s
        hg = pltpu.async_copy(g_hbm, gsc, semG)
        hb = pltpu.async_copy(be_hbm, besc, semG)
        pltpu.sync_copy(h_hbm.at[b], table.at[pl.ds(0, DH)])

        def acc_body(i, carry):
            sm, sq = carry
            v = table[pl.ds(i * L, L)]
            return (sm + v, sq + v * v)

        zero = jnp.zeros((L,), jnp.float32)
        sm, sq = plsc.parallel_loop(
            0, DH // L, unroll=4, carry=(zero, zero))(acc_body)
        tot = jnp.sum(sm)
        tot2 = jnp.sum(sq)
        mean = tot * (1.0 / DH)
        var = tot2 * (1.0 / DH) - mean * mean
        istd = _rsqrt(var + EPS)
        hg.wait()
        hb.wait()

        def n_body(i):
            v = table[pl.ds(i * L, L)]
            gv = gsc[pl.ds(i * L, L)]
            bev = besc[pl.ds(i * L, L)]
            table[pl.ds(i * L, L)] = (v - mean) * istd * gv + bev

        plsc.parallel_loop(0, DH // L, unroll=4)(n_body)
        pltpu.sync_copy(table.at[pl.ds(0, DH)], h_hbm.at[b])

    gather_layer(x, DIN, sel1, w1, b1, DH // NS, h1, False, GL=16)
    plsc.subcore_barrier()
    norm_layer(h1, g1, be1)
    plsc.subcore_barrier()
    gather_layer(h1, DH, sel2, w2, b2, DH // NS, h2, False)
    plsc.subcore_barrier()
    norm_layer(h2, g2, be2)
    plsc.subcore_barrier()
    gather_layer(h2, DH, sel3, w3, b3, DO // NS, None, True)


@jax.jit
def _run(x, sel1T, w1T, b1, g1, be1, sel2T, w2T, b2, g2, be2, sel3T, w3T, b3):
    mesh = plsc.VectorSubcoreMesh(core_axis_name="c", subcore_axis_name="s",
                                  num_cores=NC, num_subcores=NS)
    f = pl.kernel(
        _body,
        out_type=[
            jax.ShapeDtypeStruct((NB, DOUT), jnp.float32),
        ],
        mesh=mesh,
        compiler_params=pltpu.CompilerParams(needs_layout_passes=False),
        scratch_types=[
            pltpu.HBM((NB, DH), jnp.float32),
            pltpu.HBM((NB, DH), jnp.float32),
            pltpu.VMEM((G * DH,), jnp.float32),
            pltpu.VMEM((2 * PW * L,), jnp.int32),
            pltpu.VMEM((2 * PW * L,), jnp.float32),
            pltpu.VMEM((DH // NS,), jnp.float32),
            pltpu.VMEM((DH,), jnp.float32),
            pltpu.VMEM((DH,), jnp.float32),
            pltpu.VMEM((BPC * (DH // NS),), jnp.float32),
            pltpu.SemaphoreType.DMA,
            pltpu.SemaphoreType.DMA,
            pltpu.SemaphoreType.DMA,
            pltpu.SemaphoreType.DMA,
        ],
    )
    (o,) = f(x, sel1T, w1T, b1, g1, be1, sel2T, w2T, b2, g2, be2,
             sel3T, w3T, b3)
    return o


def kernel(x, sel1, w1, b1, g1, be1, sel2, w2, b2, g2, be2, sel3, w3, b3):
    sel1T = _group_major(sel1)
    w1T = _group_major(w1)
    sel2T = _group_major(sel2)
    w2T = _group_major(w2)
    sel3T = _group_major(sel3)
    w3T = _group_major(w3)
    return _run(x, sel1T, w1T, b1, g1, be1, sel2T, w2T, b2, g2, be2,
                sel3T, w3T, b3)


# speedup vs baseline: 1.0649x; 1.0025x over previous
import jax
import jax.numpy as jnp
from jax import lax
from jax.experimental import pallas as pl
from jax.experimental.pallas import tpu as pltpu
from jax.experimental.pallas import tpu_sc as plsc

NB = 32
DIN = 3200
DH = 8192
DO = 4096
PW = 128
NC = 2
NS = 16
L = 16
BPC = NB // NC
G = 8
NPASS = BPC // G
DOUT = 256
EPS = 1e-12


def _resilu(v):
    sig = 1.0 / (1.0 + jnp.exp(-v))
    return jnp.clip(v * sig, 0.0, 1.0)


def _rsqrt(x):
    i = lax.bitcast_convert_type(x, jnp.int32)
    i = jnp.int32(0x5F3759DF) - (i >> 1)
    y = lax.bitcast_convert_type(i, jnp.float32)
    for _ in range(4):
        y = y * (1.5 - 0.5 * x * y * y)
    return y


def _group_major(a):
    og = a.shape[0] // L
    return a.reshape(og, L, PW).transpose(0, 2, 1).reshape(og, PW * L)


def _body(x, sel1, w1, b1, g1, be1, sel2, w2, b2, g2, be2, sel3, w3, b3,
          out,
          h1, h2, table, idxb, wb, biasb, gsc, besc, outb,
          semA, semB, semT, semG):
    c = lax.axis_index("c")
    s = lax.axis_index("s")
    cbase = c * BPC
    sems = (semA, semB)
    BLK = PW * L

    def gather_layer(src_hbm, src_w, selT, wT, bias_hbm, U, dst_hbm, final,
                     GL=G):
        ngrp = U // L
        npass = BPC // GL
        pltpu.sync_copy(bias_hbm.at[pl.ds(s * U, U)], biasb.at[pl.ds(0, U)])

        def start_grp(g, slot):
            gg = s * ngrp + jnp.minimum(g, ngrp - 1)
            pltpu.async_copy(selT.at[gg], idxb.at[pl.ds(slot * BLK, BLK)],
                             sems[slot])
            pltpu.async_copy(wT.at[gg], wb.at[pl.ds(slot * BLK, BLK)],
                             sems[slot])

        def wait_grp(slot):
            pltpu.make_async_copy(
                selT.at[0], idxb.at[pl.ds(slot * BLK, BLK)],
                sems[slot]).wait()
            pltpu.make_async_copy(
                wT.at[0], wb.at[pl.ds(slot * BLK, BLK)], sems[slot]).wait()

        def compute_grp(g, slot):
            offi = slot * BLK
            offw = slot * BLK
            zero = jnp.zeros((L,), jnp.float32)

            def p_body(p, accs):
                idx = idxb[pl.ds(offi + p * L, L)]
                rw = _resilu(wb[pl.ds(offw + p * L, L)])
                return tuple(
                    accs[b]
                    + plsc.load_gather(table, [idx + jnp.int32(b * src_w)])
                    * rw
                    for b in range(GL))

            accs = plsc.parallel_loop(
                0, PW, unroll=2, carry=(zero,) * GL)(p_body)
            bv = biasb[pl.ds(g * L, L)]
            for b in range(GL):
                outb[pl.ds(b * U + g * L, L)] = _resilu(accs[b] - bv)

        def fire_tables(pss):
            bbase = cbase + pss * GL
            return [pltpu.async_copy(
                src_hbm.at[bbase + b, pl.ds(0, src_w)],
                table.at[pl.ds(b * src_w, src_w)], semT)
                for b in range(GL)]

        ths = fire_tables(0)
        for pss in range(npass):
            bbase = cbase + pss * GL
            start_grp(0, 0)
            for h in ths:
                h.wait()

            def two_groups(gp, _):
                g0 = 2 * gp
                start_grp(g0 + 1, 1)
                wait_grp(0)
                compute_grp(g0, 0)
                start_grp(g0 + 2, 0)
                wait_grp(1)
                compute_grp(g0 + 1, 1)
                return 0

            lax.fori_loop(0, ngrp // 2, two_groups, 0)
            wait_grp(0)
            if pss + 1 < npass:
                ths = fire_tables(pss + 1)

            if not final:
                ohs = []
                for b in range(GL):
                    ohs.append(pltpu.async_copy(
                        outb.at[pl.ds(b * U, U)],
                        dst_hbm.at[bbase + b, pl.ds(s * U, U)], semT))
                for h in ohs:
                    h.wait()
            else:
                lanes = lax.iota(jnp.int32, L)
                for b in range(GL):
                    acc = jnp.full((L,), -8.0, jnp.float32)
                    for l in range(L):
                        gi = b * U + lanes * L + l
                        acc = acc + plsc.load_gather(outb, [gi])
                    outb[pl.ds(b * U, L)] = acc
                    pltpu.sync_copy(outb.at[pl.ds(b * U, L)],
                                    out.at[bbase + b, pl.ds(s * L, L)])

    def norm_layer(h_hbm, g_hbm, be_hbm):
        b = cbase + s
        hg = pltpu.async_copy(g_hbm, gsc, semG)
        hb = pltpu.async_copy(be_hbm, besc, semG)
        pltpu.sync_copy(h_hbm.at[b], table.at[pl.ds(0, DH)])

        def acc_body(i, carry):
            sm, sq = carry
            v = table[pl.ds(i * L, L)]
            return (sm + v, sq + v * v)

        zero = jnp.zeros((L,), jnp.float32)
        sm, sq = plsc.parallel_loop(
            0, DH // L, unroll=4, carry=(zero, zero))(acc_body)
        tot = jnp.sum(sm)
        tot2 = jnp.sum(sq)
        mean = tot * (1.0 / DH)
        var = tot2 * (1.0 / DH) - mean * mean
        istd = _rsqrt(var + EPS)
        hg.wait()
        hb.wait()

        def n_body(i):
            v = table[pl.ds(i * L, L)]
            gv = gsc[pl.ds(i * L, L)]
            bev = besc[pl.ds(i * L, L)]
            table[pl.ds(i * L, L)] = (v - mean) * istd * gv + bev

        plsc.parallel_loop(0, DH // L, unroll=4)(n_body)
        pltpu.sync_copy(table.at[pl.ds(0, DH)], h_hbm.at[b])

    gather_layer(x, DIN, sel1, w1, b1, DH // NS, h1, False, GL=16)
    plsc.subcore_barrier()
    norm_layer(h1, g1, be1)
    plsc.subcore_barrier()
    gather_layer(h1, DH, sel2, w2, b2, DH // NS, h2, False)
    plsc.subcore_barrier()
    norm_layer(h2, g2, be2)
    plsc.subcore_barrier()
    gather_layer(h2, DH, sel3, w3, b3, DO // NS, None, True)


@jax.jit
def _run(x, sel1T, w1T, b1, g1, be1, sel2T, w2T, b2, g2, be2, sel3T, w3T, b3):
    mesh = plsc.VectorSubcoreMesh(core_axis_name="c", subcore_axis_name="s",
                                  num_cores=NC, num_subcores=NS)
    f = pl.kernel(
        _body,
        out_type=[
            jax.ShapeDtypeStruct((NB, DOUT), jnp.float32),
        ],
        mesh=mesh,
        compiler_params=pltpu.CompilerParams(needs_layout_passes=False),
        scratch_types=[
            pltpu.HBM((NB, DH), jnp.float32),
            pltpu.HBM((NB, DH), jnp.float32),
            pltpu.VMEM((G * DH,), jnp.float32),
            pltpu.VMEM((2 * PW * L,), jnp.int32),
            pltpu.VMEM((2 * PW * L,), jnp.float32),
            pltpu.VMEM((DH // NS,), jnp.float32),
            pltpu.VMEM((DH,), jnp.float32),
            pltpu.VMEM((DH,), jnp.float32),
            pltpu.VMEM((BPC * (DH // NS),), jnp.float32),
            pltpu.SemaphoreType.DMA,
            pltpu.SemaphoreType.DMA,
            pltpu.SemaphoreType.DMA,
            pltpu.SemaphoreType.DMA,
        ],
    )
    (o,) = f(x, sel1T, w1T, b1, g1, be1, sel2T, w2T, b2, g2, be2,
             sel3T, w3T, b3)
    return o


def kernel(x, sel1, w1, b1, g1, be1, sel2, w2, b2, g2, be2, sel3, w3, b3):
    sel1T = _group_major(sel1)
    w1T = _group_major(w1)
    sel2T = _group_major(sel2)
    w2T = _group_major(w2)
    sel3T = _group_major(sel3)
    w3T = _group_major(w3)
    return _run(x, sel1T, w1T, b1, g1, be1, sel2T, w2T, b2, g2, be2,
                sel3T, w3T, b3)
